# baseline (speedup 1.0000x reference)
import functools

import jax
import jax.numpy as jnp
from jax import lax
from jax.experimental import pallas as pl
from jax.experimental.pallas import tpu as pltpu
from jax.experimental.pallas import tpu_sc as plsc

_LANES = 16


def _xlane_sum(v):
    dnums = lax.GatherDimensionNumbers(
        offset_dims=(), collapsed_slice_dims=(0,), start_index_map=(0,))
    for k in (1, 2, 4, 8):
        idx = jnp.arange(_LANES, dtype=jnp.int32) ^ k
        v = v + lax.gather(v, idx[:, None], dnums, slice_sizes=(1,),
                           mode=lax.GatherScatterMode.PROMISE_IN_BOUNDS)
    return v


def _rsqrt_scalar(v):
    i = lax.bitcast_convert_type(v, jnp.int32)
    y = lax.bitcast_convert_type(
        jnp.int32(0x5F3759DF) - lax.shift_right_logical(i, 1), jnp.float32)
    for _ in range(3):
        y = y * (1.5 - 0.5 * v * y * y)
    return y


@functools.cache
def _make_sc_embed(BS, S, V, D, NC, NS):
    NW = NC * NS
    RPW = BS // NW
    K = 32
    NCH = RPW // K
    NJ = D // _LANES
    mesh = plsc.VectorSubcoreMesh(core_axis_name="c", subcore_axis_name="s")

    @functools.partial(
        pl.kernel,
        mesh=mesh,
        out_type=jax.ShapeDtypeStruct((BS, D), jnp.float32),
        scratch_types=[
            pltpu.VMEM((NCH, K), jnp.int32),
            pltpu.VMEM((K, D), jnp.float32),
            pltpu.VMEM((K, D), jnp.float32),
            pltpu.VMEM((D,), jnp.float32),
            pltpu.VMEM((D,), jnp.float32),
            pltpu.VMEM((_LANES,), jnp.float32),
            pltpu.SemaphoreType.DMA,
        ],
    )
    def sc_embed(ids_hbm, table_hbm, pos_hbm, gamma_hbm, beta_hbm, out_hbm,
                 idx_v, tok_v, pos_v, gamma_v, beta_v, stats_v, sem):
        wid = lax.axis_index("s") * NC + lax.axis_index("c")
        base = wid * RPW
        pos_base = lax.rem(base, S)

        pltpu.sync_copy(ids_hbm.at[wid], idx_v)
        pltpu.sync_copy(gamma_hbm, gamma_v)
        pltpu.sync_copy(beta_hbm, beta_v)

        def row_body(r, carry):
            acc = jnp.zeros((_LANES,), jnp.float32)
            acc2 = jnp.zeros((_LANES,), jnp.float32)
            for j in range(NJ):
                x = tok_v[r, pl.ds(j * _LANES, _LANES)] + pos_v[r, pl.ds(j * _LANES, _LANES)]
                acc = acc + x
                acc2 = acc2 + x * x
            vmu = _xlane_sum(acc) * (1.0 / D)
            var = _xlane_sum(acc2) * (1.0 / D) - vmu * vmu
            rinv = jnp.full((_LANES,), _rsqrt_scalar(var[0] + 1e-5), jnp.float32)
            for j in range(NJ):
                sl = pl.ds(j * _LANES, _LANES)
                x = tok_v[r, sl] + pos_v[r, sl]
                tok_v[r, sl] = (x - vmu) * rinv * gamma_v[sl] + beta_v[sl]
            return carry

        def chunk_body(c, carry):
            pltpu.async_copy(table_hbm.at[idx_v.at[c]], tok_v, sem).wait()
            pltpu.sync_copy(pos_hbm.at[pl.ds(pos_base + c * K, K)], pos_v)
            lax.fori_loop(0, K, row_body, 0)
            pltpu.sync_copy(tok_v, out_hbm.at[pl.ds(base + c * K, K)])
            return carry

        lax.fori_loop(0, NCH, chunk_body, 0)

    return sc_embed


def kernel(input_ids, token_table, pos_table, ln_gamma, ln_beta):
    B, S = input_ids.shape
    V, D = token_table.shape
    info = plsc.get_sparse_core_info()
    NC, NS = info.num_cores, info.num_subcores
    NW = NC * NS
    BS = B * S
    RPW = BS // NW
    K = 32
    ids3 = input_ids.astype(jnp.int32).reshape(NW, RPW // K, K)
    fn = _make_sc_embed(BS, S, V, D, NC, NS)
    out = fn(ids3, token_table, pos_table, ln_gamma, ln_beta)
    return out.reshape(B, S, D)


# speedup vs baseline: 1.8293x; 1.8293x over previous
import functools

import jax
import jax.numpy as jnp
from jax import lax
from jax.experimental import pallas as pl
from jax.experimental.pallas import tpu as pltpu
from jax.experimental.pallas import tpu_sc as plsc

_LANES = 16


def _xlane_sum(v):
    dnums = lax.GatherDimensionNumbers(
        offset_dims=(), collapsed_slice_dims=(0,), start_index_map=(0,))
    for k in (1, 2, 4, 8):
        idx = jnp.arange(_LANES, dtype=jnp.int32) ^ k
        v = v + lax.gather(v, idx[:, None], dnums, slice_sizes=(1,),
                           mode=lax.GatherScatterMode.PROMISE_IN_BOUNDS)
    return v


def _rsqrt_scalar(v):
    i = lax.bitcast_convert_type(v, jnp.int32)
    y = lax.bitcast_convert_type(
        jnp.int32(0x5F3759DF) - lax.shift_right_logical(i, 1), jnp.float32)
    for _ in range(3):
        y = y * (1.5 - 0.5 * v * y * y)
    return y


@functools.cache
def _make_sc_embed(B, S, V, D, NC, NS):
    NW = NC * NS
    PPW = S // NW
    NCH = 8
    PPC = PPW // NCH
    RPC = B * PPC
    NJ = D // _LANES
    mesh = plsc.VectorSubcoreMesh(core_axis_name="c", subcore_axis_name="s")

    @functools.partial(
        pl.kernel,
        mesh=mesh,
        out_type=jax.ShapeDtypeStruct((B * S, D), jnp.float32),
        scratch_types=[
            pltpu.VMEM((NCH, RPC), jnp.int32),
            pltpu.VMEM((RPC, D), jnp.float32),
            pltpu.VMEM((PPC, D), jnp.float32),
            pltpu.VMEM((D,), jnp.float32),
            pltpu.VMEM((D,), jnp.float32),
            pltpu.SemaphoreType.DMA,
        ],
    )
    def sc_embed(ids_hbm, table_hbm, pos_hbm, gamma_hbm, beta_hbm, out_hbm,
                 idx_v, tok_v, pos_v, gamma_v, beta_v, sem):
        wid = lax.axis_index("s") * NC + lax.axis_index("c")
        pos0 = wid * PPW

        pltpu.sync_copy(ids_hbm.at[wid], idx_v)
        pltpu.sync_copy(gamma_hbm, gamma_v)
        pltpu.sync_copy(beta_hbm, beta_v)

        def pos_body(p, carry):
            accs = [jnp.zeros((_LANES,), jnp.float32) for _ in range(2 * B)]
            for j in range(NJ):
                sl = pl.ds(j * _LANES, _LANES)
                vp = pos_v[p, sl]
                for b in range(B):
                    x = tok_v[b * PPC + p, sl] + vp
                    accs[2 * b] = accs[2 * b] + x
                    accs[2 * b + 1] = accs[2 * b + 1] + x * x
            scales = []
            shifts = []
            for b in range(B):
                vmu = _xlane_sum(accs[2 * b]) * (1.0 / D)
                var = _xlane_sum(accs[2 * b + 1]) * (1.0 / D) - vmu * vmu
                rinv = jnp.full((_LANES,), _rsqrt_scalar(var[0] + 1e-5),
                                jnp.float32)
                scales.append(rinv)
                shifts.append(vmu * rinv)
            for j in range(NJ):
                sl = pl.ds(j * _LANES, _LANES)
                vp = pos_v[p, sl]
                g = gamma_v[sl]
                be = beta_v[sl]
                for b in range(B):
                    x = tok_v[b * PPC + p, sl] + vp
                    tok_v[b * PPC + p, sl] = (x * scales[b] - shifts[b]) * g + be
            return carry

        def chunk_body(c, carry):
            pltpu.async_copy(table_hbm.at[idx_v.at[c]], tok_v, sem).wait()
            pltpu.sync_copy(pos_hbm.at[pl.ds(pos0 + c * PPC, PPC)], pos_v)
            lax.fori_loop(0, PPC, pos_body, 0)
            for b in range(B):
                pltpu.sync_copy(
                    tok_v.at[pl.ds(b * PPC, PPC)],
                    out_hbm.at[pl.ds(b * S + pos0 + c * PPC, PPC)])
            return carry

        lax.fori_loop(0, NCH, chunk_body, 0)

    return sc_embed


def kernel(input_ids, token_table, pos_table, ln_gamma, ln_beta):
    B, S = input_ids.shape
    V, D = token_table.shape
    info = plsc.get_sparse_core_info()
    NC, NS = info.num_cores, info.num_subcores
    NW = NC * NS
    NCH = 8
    PPC = S // NW // NCH
    ids3 = (input_ids.astype(jnp.int32)
            .reshape(B, NW, NCH, PPC)
            .transpose(1, 2, 0, 3)
            .reshape(NW, NCH, B * PPC))
    fn = _make_sc_embed(B, S, V, D, NC, NS)
    out = fn(ids3, token_table, pos_table, ln_gamma, ln_beta)
    return out.reshape(B, S, D)
